# baseline (speedup 1.0000x reference)
import functools

import jax
import jax.numpy as jnp
from jax import lax
from jax.experimental import pallas as pl
from jax.experimental.pallas import tpu as pltpu
from jax.experimental.pallas import tpu_sc as plsc

CAP = 65536
X_DIM = 128
Y0, Y1 = 32, 32

GRID = 32
CH = CAP // GRID


def _cam_body(pos_ref, crow_ref, cb_in, cb_out):
    i = pl.program_id(0)
    base = i * CH
    pos = pos_ref[0]
    local = pos - base
    in_range = (pos >= base) & (pos < base + CH)

    @pl.when(in_range)
    def _cam_sel():
        lane = jax.lax.broadcasted_iota(jnp.int32, (Y0, Y1, CH), 2)
        cb_out[...] = jnp.where(lane == local, crow_ref[...], cb_in[...])

    @pl.when(jnp.logical_not(in_range))
    def _cam_copy():
        cb_out[...] = cb_in[...]


NC = 2
NS = 16
NW = NC * NS

ST_TROWS = CAP // NW
SR_S = 256
NBUF_S = 3
ST_STRIPES = ST_TROWS // SR_S

IT_CHUNK = 16384
IT_STRIPES = CAP // IT_CHUNK


def _sc_body(pos16_h, cnt16_h, srow_h, sb_h, it_h, sb_o, it_o,
             st_buf, it_buf, srow_v, posv, cntv,
             sem_si, sem_so, sem_it, sem_row):
    wid = lax.axis_index("s") * NC + lax.axis_index("c")

    pltpu.async_copy(pos16_h, posv, sem_row).wait()
    pltpu.async_copy(cnt16_h, cntv, sem_row).wait()
    pos = posv[...][0]
    cnt = cntv[...][0]

    st_base = wid * ST_TROWS
    sins = {}
    souts = {}

    def st_in(s):
        b = s % NBUF_S
        sins[s] = pltpu.async_copy(
            sb_h.at[pl.ds(st_base + s * SR_S, SR_S)], st_buf.at[b],
            sem_si.at[b])

    def st_out(s):
        b = s % NBUF_S
        souts[s] = pltpu.async_copy(
            st_buf.at[b], sb_o.at[pl.ds(st_base + s * SR_S, SR_S)],
            sem_so.at[b])

    for s in range(min(NBUF_S, ST_STRIPES)):
        st_in(s)
    for s in range(ST_STRIPES):
        sins[s].wait()
        st_out(s)
        t = s - 1
        if t >= 0 and t + NBUF_S < ST_STRIPES:
            souts[t].wait()
            st_in(t + NBUF_S)
    swaited = set(range(0, max(0, ST_STRIPES - NBUF_S)))
    for s in range(ST_STRIPES):
        if s not in swaited:
            souts[s].wait()

    @pl.when(wid == pos // ST_TROWS)
    def _st_row():
        pltpu.async_copy(srow_h, srow_v, sem_row).wait()
        pltpu.async_copy(srow_v, sb_o.at[pl.ds(pos, 1)], sem_row).wait()

    @pl.when(wid == 0)
    def _iter():
        for s in range(IT_STRIPES):
            base = s * IT_CHUNK
            pltpu.async_copy(it_h.at[pl.ds(base, IT_CHUNK)], it_buf,
                             sem_it).wait()

            @pl.when((pos >= base) & (pos < base + IT_CHUNK))
            def _patch(base=base):
                local = pos - base
                off = (local // 16) * 16
                lane = local - off
                v = it_buf[pl.ds(off, 16)]
                w = jnp.where(lax.iota(jnp.int32, 16) == lane, cnt, v)
                it_buf[pl.ds(off, 16)] = w

            pltpu.async_copy(it_buf, it_o.at[pl.ds(base, IT_CHUNK)],
                             sem_it).wait()


def _sc_push(state_buffer, iter_buffer, pos16, cnt16, srow):
    mesh = plsc.VectorSubcoreMesh(core_axis_name="c", subcore_axis_name="s")
    f = pl.kernel(
        _sc_body,
        out_type=[
            jax.ShapeDtypeStruct((CAP, X_DIM), jnp.float32),
            jax.ShapeDtypeStruct((CAP,), jnp.int32),
        ],
        mesh=mesh,
        scratch_types=[
            pltpu.VMEM((NBUF_S, SR_S, X_DIM), jnp.float32),
            pltpu.VMEM((IT_CHUNK,), jnp.int32),
            pltpu.VMEM((1, X_DIM), jnp.float32),
            pltpu.VMEM((16,), jnp.int32),
            pltpu.VMEM((16,), jnp.int32),
            pltpu.SemaphoreType.DMA((NBUF_S,)),
            pltpu.SemaphoreType.DMA((NBUF_S,)),
            pltpu.SemaphoreType.DMA,
            pltpu.SemaphoreType.DMA,
        ],
    )
    return f(pos16, cnt16, srow, state_buffer, iter_buffer)


def kernel(state_buffer, cam_data_buffer, iter_buffer, position, state,
           cam_data, count):
    pos2 = position.reshape(1)
    pos16 = jnp.broadcast_to(position, (16,)).astype(jnp.int32)
    cnt16 = jnp.broadcast_to(count, (16,)).astype(jnp.int32)
    srow = state.reshape(1, X_DIM)
    crow = cam_data.reshape(Y0, Y1, 1)
    cam_t = jax.lax.transpose(cam_data_buffer, (1, 2, 0))

    out_cb = pl.pallas_call(
        _cam_body,
        grid=(GRID,),
        in_specs=[
            pl.BlockSpec(memory_space=pltpu.SMEM),
            pl.BlockSpec((Y0, Y1, 1), lambda i: (0, 0, 0)),
            pl.BlockSpec((Y0, Y1, CH), lambda i: (0, 0, i)),
        ],
        out_specs=pl.BlockSpec((Y0, Y1, CH), lambda i: (0, 0, i)),
        out_shape=jax.ShapeDtypeStruct((Y0, Y1, CAP), jnp.float32),
        compiler_params=pltpu.CompilerParams(
            dimension_semantics=("arbitrary",),
        ),
    )(pos2, crow, cam_t)

    out_sb, out_it = _sc_push(state_buffer, iter_buffer, pos16, cnt16, srow)

    new_position = jnp.remainder(position + 1, CAP)
    full_buffer = (position + 1) == CAP
    return (out_sb, jax.lax.transpose(out_cb, (2, 0, 1)), out_it,
            new_position, full_buffer)


# speedup vs baseline: 1.0840x; 1.0840x over previous
import jax
import jax.numpy as jnp
from jax.experimental import pallas as pl
from jax.experimental.pallas import tpu as pltpu

CAP = 65536
X_DIM = 128
Y0, Y1 = 32, 32

GRID = 32
CH = CAP // GRID


def _push_body(pos_ref, cnt_ref, srow_ref, crow_ref, sb_in, cb_in, it_in,
               sb_out, cb_out, it_out):
    i = pl.program_id(0)
    base = i * CH
    pos = pos_ref[0]
    cnt = cnt_ref[0]
    local = pos - base
    in_range = (pos >= base) & (pos < base + CH)

    sb_out[...] = sb_in[...]

    @pl.when(in_range)
    def _cam_sel():
        lane = jax.lax.broadcasted_iota(jnp.int32, (Y0, Y1, CH), 2)
        crow3 = crow_ref[...][:, :, None]
        cb_out[...] = jnp.where(lane == local, crow3, cb_in[...])

    @pl.when(jnp.logical_not(in_range))
    def _cam_copy():
        cb_out[...] = cb_in[...]

    it_out[...] = it_in[...]

    @pl.when(in_range)
    def _overwrite():
        sb_out[pl.ds(local, 1), :] = srow_ref[...]
        col = jax.lax.broadcasted_iota(jnp.int32, (1, 1, CH), 2)
        it_out[...] = jnp.where(col == local, cnt, it_in[...])


def kernel(state_buffer, cam_data_buffer, iter_buffer, position, state,
           cam_data, count):
    pos2 = position.reshape(1)
    cnt2 = count.reshape(1)
    srow = state.reshape(1, X_DIM)
    crow = cam_data
    cam_t = jax.lax.transpose(cam_data_buffer, (1, 2, 0))
    iter3d = iter_buffer.reshape(GRID, 1, CH)

    out_sb, out_cb, out_it = pl.pallas_call(
        _push_body,
        grid=(GRID,),
        in_specs=[
            pl.BlockSpec(memory_space=pltpu.SMEM),
            pl.BlockSpec(memory_space=pltpu.SMEM),
            pl.BlockSpec((1, X_DIM), lambda i: (0, 0)),
            pl.BlockSpec((Y0, Y1), lambda i: (0, 0)),
            pl.BlockSpec((CH, X_DIM), lambda i: (i, 0)),
            pl.BlockSpec((Y0, Y1, CH), lambda i: (0, 0, i)),
            pl.BlockSpec((1, 1, CH), lambda i: (i, 0, 0)),
        ],
        out_specs=[
            pl.BlockSpec((CH, X_DIM), lambda i: (i, 0)),
            pl.BlockSpec((Y0, Y1, CH), lambda i: (0, 0, i)),
            pl.BlockSpec((1, 1, CH), lambda i: (i, 0, 0)),
        ],
        out_shape=[
            jax.ShapeDtypeStruct((CAP, X_DIM), jnp.float32),
            jax.ShapeDtypeStruct((Y0, Y1, CAP), jnp.float32),
            jax.ShapeDtypeStruct((GRID, 1, CH), jnp.int32),
        ],
        compiler_params=pltpu.CompilerParams(
            dimension_semantics=("arbitrary",),
        ),
    )(pos2, cnt2, srow, crow, state_buffer, cam_t, iter3d)

    new_position = jnp.remainder(position + 1, CAP)
    full_buffer = (position + 1) == CAP
    return (out_sb, jax.lax.transpose(out_cb, (2, 0, 1)),
            out_it.reshape(CAP), new_position, full_buffer)


# speedup vs baseline: 1.1096x; 1.0236x over previous
import jax
import jax.numpy as jnp
from jax.experimental import pallas as pl
from jax.experimental.pallas import tpu as pltpu

CAP = 65536
X_DIM = 128
Y0, Y1 = 32, 32

GRID = 16
CH = CAP // GRID
SRCH = CAP // (2 * GRID)


def _push_body(pos_ref, cnt_ref, srow_ref, crow_ref, sb_in, cb_in, it_in,
               sb_out, cb_out, it_out):
    j = pl.program_id(0)
    i = pl.program_id(1)
    pos = pos_ref[0]
    cnt = cnt_ref[0]

    cbase = i * CH
    clocal = pos - cbase
    cam_in_range = (pos >= cbase) & (pos < cbase + CH)

    @pl.when(cam_in_range)
    def _cam_sel():
        lane = jax.lax.broadcasted_iota(jnp.int32, (Y0 // 2, Y1, CH), 2)
        crow3 = crow_ref[...][:, :, None]
        cb_out[...] = jnp.where(lane == clocal, crow3, cb_in[...])

    @pl.when(jnp.logical_not(cam_in_range))
    def _cam_copy():
        cb_out[...] = cb_in[...]

    sb_out[...] = sb_in[...]
    it_out[...] = it_in[...]

    sbase = (i * 2 + j) * SRCH
    slocal = pos - sbase

    @pl.when((pos >= sbase) & (pos < sbase + SRCH))
    def _overwrite():
        sb_out[pl.ds(slocal, 1), :] = srow_ref[...]
        col = jax.lax.broadcasted_iota(jnp.int32, (1, 1, SRCH), 2)
        it_out[...] = jnp.where(col == slocal, cnt, it_in[...])


def kernel(state_buffer, cam_data_buffer, iter_buffer, position, state,
           cam_data, count):
    pos2 = position.reshape(1)
    cnt2 = count.reshape(1)
    srow = state.reshape(1, X_DIM)
    crow = cam_data
    cam_t = jax.lax.transpose(cam_data_buffer, (1, 2, 0))
    iter3d = iter_buffer.reshape(2 * GRID, 1, SRCH)

    out_sb, out_cb, out_it = pl.pallas_call(
        _push_body,
        grid=(2, GRID),
        in_specs=[
            pl.BlockSpec(memory_space=pltpu.SMEM),
            pl.BlockSpec(memory_space=pltpu.SMEM),
            pl.BlockSpec((1, X_DIM), lambda j, i: (0, 0)),
            pl.BlockSpec((Y0 // 2, Y1), lambda j, i: (j, 0)),
            pl.BlockSpec((SRCH, X_DIM), lambda j, i: (i * 2 + j, 0)),
            pl.BlockSpec((Y0 // 2, Y1, CH), lambda j, i: (j, 0, i)),
            pl.BlockSpec((1, 1, SRCH), lambda j, i: (i * 2 + j, 0, 0)),
        ],
        out_specs=[
            pl.BlockSpec((SRCH, X_DIM), lambda j, i: (i * 2 + j, 0)),
            pl.BlockSpec((Y0 // 2, Y1, CH), lambda j, i: (j, 0, i)),
            pl.BlockSpec((1, 1, SRCH), lambda j, i: (i * 2 + j, 0, 0)),
        ],
        out_shape=[
            jax.ShapeDtypeStruct((CAP, X_DIM), jnp.float32),
            jax.ShapeDtypeStruct((Y0, Y1, CAP), jnp.float32),
            jax.ShapeDtypeStruct((2 * GRID, 1, SRCH), jnp.int32),
        ],
        compiler_params=pltpu.CompilerParams(
            dimension_semantics=("arbitrary", "arbitrary"),
        ),
    )(pos2, cnt2, srow, crow, state_buffer, cam_t, iter3d)

    new_position = jnp.remainder(position + 1, CAP)
    full_buffer = (position + 1) == CAP
    return (out_sb, jax.lax.transpose(out_cb, (2, 0, 1)),
            out_it.reshape(CAP), new_position, full_buffer)
